# baseline (speedup 1.0000x reference)
import functools

import jax
import jax.numpy as jnp
from jax import lax
from jax.experimental import pallas as pl
from jax.experimental.pallas import tpu as pltpu
from jax.experimental.pallas import tpu_sc as plsc

L = 16
NC = 2
NS = 16
NW = NC * NS
UNROLL = 8
HALVES = 2


def _insert(m1, m2, m3, x):
    n1 = jnp.maximum(m1, x)
    s1 = jnp.minimum(m1, x)
    n2 = jnp.maximum(m2, s1)
    s2 = jnp.minimum(m2, s1)
    n3 = jnp.maximum(m3, s2)
    return n1, n2, n3


def _bfly_max(x):
    idx = lax.iota(jnp.int32, L)
    for s in (8, 4, 2, 1):
        x = jnp.maximum(x, jnp.take_along_axis(x, idx ^ s, axis=0))
    return x


def _bfly_min_i32(x):
    idx = lax.iota(jnp.int32, L)
    for s in (8, 4, 2, 1):
        x = jnp.minimum(x, jnp.take_along_axis(x, idx ^ s, axis=0))
    return x


def _remove_first(m1, m2, m3, t):
    lane = lax.iota(jnp.int32, L)
    cand = jnp.where(m1 == t, lane, L)
    first = _bfly_min_i32(cand)
    oh = lane == first
    return jnp.where(oh, m2, m1), jnp.where(oh, m3, m2)


def _top3(m1, m2, m3):
    t1 = _bfly_max(m1)
    m1, m2 = _remove_first(m1, m2, m3, t1)
    t2 = _bfly_max(m1)
    m1, _ = _remove_first(m1, m2, m3, t2)
    t3 = _bfly_max(m1)
    return t1, t2, t3


TC_ROWS = 64


def _tc_partial(output, target):
    _, N = output.shape
    groups = TC_ROWS // 8

    def body(o_ref, t_ref, out_ref):
        neg = jnp.full((8, 128), -jnp.inf, jnp.float32)

        def tops(ref):
            def step(j, c):
                m1, m2, m3, = c
                v = ref[:, pl.ds(j * 128, 128)]
                n1 = jnp.maximum(m1, v)
                s1 = jnp.minimum(m1, v)
                n2 = jnp.maximum(m2, s1)
                s2 = jnp.minimum(m2, s1)
                n3 = jnp.maximum(m3, s2)
                return n1, n2, n3

            m1, m2, m3 = lax.fori_loop(0, N // 128, step, (neg, neg, neg))
            li = lax.broadcasted_iota(jnp.int32, (8, 128), 1)

            def remove_first(m1, m2, m3, t):
                cand = jnp.where(m1 == t, li, 128)
                fl = jnp.min(cand, axis=1, keepdims=True)
                oh = li == fl
                return jnp.where(oh, m2, m1), jnp.where(oh, m3, m2)

            t1 = jnp.max(m1, axis=1, keepdims=True)
            m1, m2 = remove_first(m1, m2, m3, t1)
            t2 = jnp.max(m1, axis=1, keepdims=True)
            m1, _ = remove_first(m1, m2, m3, t2)
            t3 = jnp.max(m1, axis=1, keepdims=True)
            return t1, t2, t3

        a1, a2, a3 = tops(o_ref)
        b1, b2, b3 = tops(t_ref)
        d1 = a1 - b1
        d2 = a2 - b2
        d3 = a3 - b3
        part = jnp.sum(9.0 * d1 * d1 + 4.0 * d2 * d2 + d3 * d3)

        @pl.when(pl.program_id(0) == 0)
        def _():
            out_ref[0, 0] = 0.0

        out_ref[0, 0] += part

    return pl.pallas_call(
        body,
        grid=(groups,),
        in_specs=[
            pl.BlockSpec((8, N), lambda j: (j, 0)),
            pl.BlockSpec((8, N), lambda j: (j, 0)),
        ],
        out_specs=pl.BlockSpec(memory_space=pltpu.SMEM),
        out_shape=jax.ShapeDtypeStruct((1, 1), jnp.float32),
    )(output, target)


def kernel(output, target):
    R, N = output.shape
    rows_sc = R - TC_ROWS
    rows_per = rows_sc // NW
    ch = N // HALVES
    steps = ch // (L * UNROLL)
    nchunks = rows_per * HALVES

    mesh = plsc.VectorSubcoreMesh(core_axis_name="c", subcore_axis_name="s")

    @functools.partial(
        pl.kernel,
        mesh=mesh,
        out_type=jax.ShapeDtypeStruct((NW, L), jnp.float32),
        scratch_types=[
            pltpu.VMEM((ch,), jnp.float32),
            pltpu.VMEM((ch,), jnp.float32),
            pltpu.VMEM((ch,), jnp.float32),
            pltpu.VMEM((ch,), jnp.float32),
            pltpu.VMEM((L,), jnp.float32),
            pltpu.SemaphoreType.DMA,
            pltpu.SemaphoreType.DMA,
        ],
        compiler_params=pltpu.CompilerParams(needs_layout_passes=False),
    )
    def sc_loss(out_hbm, tgt_hbm, part_hbm, bo0, bt0, bo1, bt1, buf_p, s0, s1):
        wid = lax.axis_index("s") * NC + lax.axis_index("c")
        slots = [(bo0, bt0, s0), (bo1, bt1, s1)]
        neg = jnp.full((L,), -jnp.inf, jnp.float32)
        row0 = TC_ROWS + wid * rows_per

        def copies(row, h):
            bo, bt, sem = slots[h]
            off = h * ch
            return (pltpu.make_async_copy(out_hbm.at[row, pl.ds(off, ch)], bo, sem),
                    pltpu.make_async_copy(tgt_hbm.at[row, pl.ds(off, ch)], bt, sem))

        for h in range(HALVES):
            for c in copies(row0, h):
                c.start()

        def row_body(r, acc):
            row = row0 + r
            carry = (neg, neg, neg, neg, neg, neg)
            for h in range(HALVES):
                for c in copies(row, h):
                    c.wait()
                bo, bt, _ = slots[h]

                def step(i, c, bo=bo, bt=bt):
                    o1, o2, o3, t1, t2, t3 = c
                    for u in range(UNROLL):
                        base = (i * UNROLL + u) * L
                        x = bo[pl.ds(base, L)]
                        y = bt[pl.ds(base, L)]
                        o1, o2, o3 = _insert(o1, o2, o3, x)
                        t1, t2, t3 = _insert(t1, t2, t3, y)
                    return o1, o2, o3, t1, t2, t3

                carry = lax.fori_loop(0, steps, step, carry)

                @pl.when(r + 1 < rows_per)
                def _():
                    for c in copies(row + 1, h):
                        c.start()

            o1, o2, o3, t1, t2, t3 = carry
            a1, a2, a3 = _top3(o1, o2, o3)
            b1, b2, b3 = _top3(t1, t2, t3)
            d1 = a1 - b1
            d2 = a2 - b2
            d3 = a3 - b3
            return acc + 9.0 * d1 * d1 + 4.0 * d2 * d2 + d3 * d3

        acc = lax.fori_loop(0, rows_per, row_body, jnp.zeros((L,), jnp.float32))
        buf_p[...] = acc
        pltpu.sync_copy(buf_p, part_hbm.at[wid])

    parts = sc_loss(output, target)
    tc_part = _tc_partial(output, target)
    return (jnp.sum(parts[:, 0]) + tc_part[0, 0]) / jnp.float32(R * 3)


# speedup vs baseline: 1.4679x; 1.4679x over previous
import functools

import jax
import jax.numpy as jnp
from jax import lax
from jax.experimental import pallas as pl
from jax.experimental.pallas import tpu as pltpu
from jax.experimental.pallas import tpu_sc as plsc

L = 16
NC = 2
NS = 16
NW = NC * NS
UNROLL = 8
HALVES = 2


def _insert(m1, m2, m3, x):
    n1 = jnp.maximum(m1, x)
    s1 = jnp.minimum(m1, x)
    n2 = jnp.maximum(m2, s1)
    s2 = jnp.minimum(m2, s1)
    n3 = jnp.maximum(m3, s2)
    return n1, n2, n3


def _bfly_max(x):
    idx = lax.iota(jnp.int32, L)
    for s in (8, 4, 2, 1):
        x = jnp.maximum(x, jnp.take_along_axis(x, idx ^ s, axis=0))
    return x


def _bfly_min_i32(x):
    idx = lax.iota(jnp.int32, L)
    for s in (8, 4, 2, 1):
        x = jnp.minimum(x, jnp.take_along_axis(x, idx ^ s, axis=0))
    return x


def _remove_first(m1, m2, m3, t):
    lane = lax.iota(jnp.int32, L)
    cand = jnp.where(m1 == t, lane, L)
    first = _bfly_min_i32(cand)
    oh = lane == first
    return jnp.where(oh, m2, m1), jnp.where(oh, m3, m2)


def _top3(m1, m2, m3):
    t1 = _bfly_max(m1)
    m1, m2 = _remove_first(m1, m2, m3, t1)
    t2 = _bfly_max(m1)
    m1, _ = _remove_first(m1, m2, m3, t2)
    t3 = _bfly_max(m1)
    return t1, t2, t3


TC_ROWS = 32


def _tc_partial(output, target):
    _, N = output.shape
    groups = TC_ROWS // 8

    def body(o_ref, t_ref, out_ref):
        neg = jnp.full((8, 128), -jnp.inf, jnp.float32)

        def tops(ref):
            def step(j, c):
                m1, m2, m3, = c
                v = ref[:, pl.ds(j * 128, 128)]
                n1 = jnp.maximum(m1, v)
                s1 = jnp.minimum(m1, v)
                n2 = jnp.maximum(m2, s1)
                s2 = jnp.minimum(m2, s1)
                n3 = jnp.maximum(m3, s2)
                return n1, n2, n3

            m1, m2, m3 = lax.fori_loop(0, N // 128, step, (neg, neg, neg))
            li = lax.broadcasted_iota(jnp.int32, (8, 128), 1)

            def remove_first(m1, m2, m3, t):
                cand = jnp.where(m1 == t, li, 128)
                fl = jnp.min(cand, axis=1, keepdims=True)
                oh = li == fl
                return jnp.where(oh, m2, m1), jnp.where(oh, m3, m2)

            t1 = jnp.max(m1, axis=1, keepdims=True)
            m1, m2 = remove_first(m1, m2, m3, t1)
            t2 = jnp.max(m1, axis=1, keepdims=True)
            m1, _ = remove_first(m1, m2, m3, t2)
            t3 = jnp.max(m1, axis=1, keepdims=True)
            return t1, t2, t3

        a1, a2, a3 = tops(o_ref)
        b1, b2, b3 = tops(t_ref)
        d1 = a1 - b1
        d2 = a2 - b2
        d3 = a3 - b3
        part = jnp.sum(9.0 * d1 * d1 + 4.0 * d2 * d2 + d3 * d3)

        @pl.when(pl.program_id(0) == 0)
        def _():
            out_ref[0, 0] = 0.0

        out_ref[0, 0] += part

    return pl.pallas_call(
        body,
        grid=(groups,),
        in_specs=[
            pl.BlockSpec((8, N), lambda j: (j, 0)),
            pl.BlockSpec((8, N), lambda j: (j, 0)),
        ],
        out_specs=pl.BlockSpec(memory_space=pltpu.SMEM),
        out_shape=jax.ShapeDtypeStruct((1, 1), jnp.float32),
    )(output, target)


def kernel(output, target):
    R, N = output.shape
    rows_sc = R - TC_ROWS
    rows_per = rows_sc // NW
    ch = N // HALVES
    steps = ch // (L * UNROLL)
    nchunks = rows_per * HALVES

    mesh = plsc.VectorSubcoreMesh(core_axis_name="c", subcore_axis_name="s")

    @functools.partial(
        pl.kernel,
        mesh=mesh,
        out_type=jax.ShapeDtypeStruct((NW, L), jnp.float32),
        scratch_types=[
            pltpu.VMEM((ch,), jnp.float32),
            pltpu.VMEM((ch,), jnp.float32),
            pltpu.VMEM((ch,), jnp.float32),
            pltpu.VMEM((ch,), jnp.float32),
            pltpu.VMEM((L,), jnp.float32),
            pltpu.SemaphoreType.DMA,
            pltpu.SemaphoreType.DMA,
        ],
        compiler_params=pltpu.CompilerParams(needs_layout_passes=False),
    )
    def sc_loss(out_hbm, tgt_hbm, part_hbm, bo0, bt0, bo1, bt1, buf_p, s0, s1):
        wid = lax.axis_index("s") * NC + lax.axis_index("c")
        slots = [(bo0, bt0, s0), (bo1, bt1, s1)]
        neg = jnp.full((L,), -jnp.inf, jnp.float32)
        row0 = TC_ROWS + wid * rows_per

        def copies(row, h):
            bo, bt, sem = slots[h]
            off = h * ch
            return (pltpu.make_async_copy(out_hbm.at[row, pl.ds(off, ch)], bo, sem),
                    pltpu.make_async_copy(tgt_hbm.at[row, pl.ds(off, ch)], bt, sem))

        for h in range(HALVES):
            for c in copies(row0, h):
                c.start()

        def row_body(r, acc):
            row = row0 + r
            carry = (neg, neg, neg, neg, neg, neg)
            for h in range(HALVES):
                for c in copies(row, h):
                    c.wait()
                bo, bt, _ = slots[h]

                def step(i, c, bo=bo, bt=bt):
                    o1, o2, o3, t1, t2, t3 = c
                    for u in range(UNROLL):
                        base = (i * UNROLL + u) * L
                        x = bo[pl.ds(base, L)]
                        y = bt[pl.ds(base, L)]
                        o1, o2, o3 = _insert(o1, o2, o3, x)
                        t1, t2, t3 = _insert(t1, t2, t3, y)
                    return o1, o2, o3, t1, t2, t3

                carry = lax.fori_loop(0, steps, step, carry)

                @pl.when(r + 1 < rows_per)
                def _():
                    for c in copies(row + 1, h):
                        c.start()

            o1, o2, o3, t1, t2, t3 = carry
            a1, a2, a3 = _top3(o1, o2, o3)
            b1, b2, b3 = _top3(t1, t2, t3)
            d1 = a1 - b1
            d2 = a2 - b2
            d3 = a3 - b3
            return acc + 9.0 * d1 * d1 + 4.0 * d2 * d2 + d3 * d3

        acc = lax.fori_loop(0, rows_per, row_body, jnp.zeros((L,), jnp.float32))
        buf_p[...] = acc
        pltpu.sync_copy(buf_p, part_hbm.at[wid])

    parts = sc_loss(output, target)
    tc_part = _tc_partial(output, target)
    return (jnp.sum(parts[:, 0]) + tc_part[0, 0]) / jnp.float32(R * 3)
